# baseline (speedup 1.0000x reference)
import functools

import jax
import jax.numpy as jnp
import numpy as np
from jax import lax
from jax.experimental import pallas as pl
from jax.experimental.pallas import tpu as pltpu
from jax.experimental.pallas import tpu_sc as plsc

D_MODEL = 768
LANES = 16
NGRP = D_MODEL // (2 * LANES)


@functools.lru_cache(maxsize=None)
def _pos_enc(seq_len):
    pos = np.arange(seq_len, dtype=np.float32)[:, None]
    _2i = np.arange(0, D_MODEL, 2, dtype=np.float32)
    enc = np.zeros((seq_len, D_MODEL), dtype=np.float32)
    enc[:, 0::2] = np.sin(pos / np.float32(10000.0) ** (_2i / D_MODEL))
    enc[:, 1::2] = np.cos(pos / np.float32(10000.0) ** (_2i / D_MODEL))
    return enc.reshape(-1)


@functools.lru_cache(maxsize=None)
def _make_kernel(batch, seq):
    B = batch * seq
    info = plsc.get_sparse_core_info()
    NC, NS = info.num_cores, info.num_subcores
    NW = NC * NS
    b_per_w = B // NW
    C = 32
    n_chunks = b_per_w // C

    mesh = plsc.VectorSubcoreMesh(core_axis_name="c", subcore_axis_name="s")

    @functools.partial(
        pl.kernel,
        mesh=mesh,
        out_type=jax.ShapeDtypeStruct((batch, seq, D_MODEL), jnp.float32),
        scratch_types=[
            pltpu.VMEM((b_per_w,), jnp.int32),
            pltpu.VMEM((C, D_MODEL), jnp.float32),
            pltpu.VMEM((C, D_MODEL), jnp.float32),
            pltpu.VMEM((C * D_MODEL,), jnp.float32),
            pltpu.VMEM((C * D_MODEL,), jnp.float32),
            pltpu.SemaphoreType.DMA,
            pltpu.SemaphoreType.DMA,
            pltpu.SemaphoreType.DMA,
            pltpu.SemaphoreType.DMA,
            pltpu.SemaphoreType.DMA,
            pltpu.SemaphoreType.DMA,
        ],
    )
    def emb_kernel(idx_hbm, table_hbm, enc_hbm, out_hbm,
                   idx_v, rows0, rows1, enc0, enc1,
                   sg0, sg1, se0, se1, ss0, ss1):
        wid = lax.axis_index("s") * NC + lax.axis_index("c")
        base = wid * b_per_w
        b = base // seq
        col = base % seq
        pltpu.sync_copy(idx_hbm.at[b, pl.ds(col, b_per_w)], idx_v)

        rows = (rows0, rows1)
        encs = (enc0, enc1)
        gsem = (sg0, sg1)
        esem = (se0, se1)
        ssem = (ss0, ss1)
        gops = [None, None]
        eops = [None, None]
        sops = [None, None]

        def issue(c):
            p = c & 1
            gops[p] = pltpu.async_copy(
                table_hbm.at[idx_v.at[pl.ds(c * C, C)]], rows[p], gsem[p])
            eops[p] = pltpu.async_copy(
                enc_hbm.at[pl.ds((col + c * C) * D_MODEL, C * D_MODEL)],
                encs[p], esem[p])

        issue(0)
        issue(1)
        for c in range(n_chunks):
            p = c & 1
            gops[p].wait()
            eops[p].wait()

            def row_body(r, _, p=p):
                rb = r * D_MODEL
                for j in range(D_MODEL // LANES):
                    o = j * LANES
                    rows[p][r, pl.ds(o, LANES)] = (
                        rows[p][r, pl.ds(o, LANES)]
                        + encs[p][pl.ds(rb + o, LANES)])
                return 0

            lax.fori_loop(0, C, row_body, 0)
            sops[p] = pltpu.async_copy(
                rows[p], out_hbm.at[b, pl.ds(col + c * C, C)], ssem[p])
            if c + 2 < n_chunks:
                sops[p].wait()
                issue(c + 2)
        sops[(n_chunks - 2) & 1].wait()
        sops[(n_chunks - 1) & 1].wait()

    return emb_kernel


def kernel(x, table):
    batch, seq = x.shape
    enc = jnp.asarray(_pos_enc(seq))
    return _make_kernel(batch, seq)(x.astype(jnp.int32), table, enc)


# speedup vs baseline: 1.7011x; 1.7011x over previous
import functools

import jax
import jax.numpy as jnp
import numpy as np
from jax import lax
from jax.experimental import pallas as pl
from jax.experimental.pallas import tpu as pltpu
from jax.experimental.pallas import tpu_sc as plsc

D_MODEL = 768
LANES = 16


@functools.lru_cache(maxsize=None)
def _pos_enc(seq_len):
    pos = np.arange(seq_len, dtype=np.float32)[:, None]
    _2i = np.arange(0, D_MODEL, 2, dtype=np.float32)
    enc = np.zeros((seq_len, D_MODEL), dtype=np.float32)
    enc[:, 0::2] = np.sin(pos / np.float32(10000.0) ** (_2i / D_MODEL))
    enc[:, 1::2] = np.cos(pos / np.float32(10000.0) ** (_2i / D_MODEL))
    return enc


@functools.lru_cache(maxsize=None)
def _make_kernel(batch, seq):
    B = batch * seq
    info = plsc.get_sparse_core_info()
    NC, NS = info.num_cores, info.num_subcores
    NW = NC * NS
    b_per_w = B // NW
    C = 32
    n_chunks = b_per_w // C

    mesh = plsc.VectorSubcoreMesh(core_axis_name="c", subcore_axis_name="s")

    @functools.partial(
        pl.kernel,
        mesh=mesh,
        out_type=jax.ShapeDtypeStruct((batch, seq, D_MODEL), jnp.float32),
        scratch_types=[
            pltpu.VMEM((b_per_w,), jnp.int32),
            pltpu.VMEM((C, D_MODEL), jnp.float32),
            pltpu.VMEM((C, D_MODEL), jnp.float32),
            pltpu.VMEM((C, D_MODEL), jnp.float32),
            pltpu.VMEM((C, D_MODEL), jnp.float32),
            pltpu.VMEM((C, D_MODEL), jnp.float32),
            pltpu.SemaphoreType.DMA,
            pltpu.SemaphoreType.DMA,
            pltpu.SemaphoreType.DMA,
            pltpu.SemaphoreType.DMA,
            pltpu.SemaphoreType.DMA,
            pltpu.SemaphoreType.DMA,
            pltpu.SemaphoreType.DMA,
            pltpu.SemaphoreType.DMA,
        ],
    )
    def emb_kernel(idx_hbm, table_hbm, enc_hbm, out_hbm,
                   idx_v, rows0, rows1, rows2, enc0, enc1,
                   sg0, sg1, sg2, se0, se1, ss0, ss1, ss2):
        wid = lax.axis_index("s") * NC + lax.axis_index("c")
        base = wid * b_per_w
        b = base // seq
        col = base % seq
        pltpu.sync_copy(idx_hbm.at[b, pl.ds(col, b_per_w)], idx_v)

        rows = (rows0, rows1, rows2)
        encs = (enc0, enc1)
        gsem = (sg0, sg1, sg2)
        esem = (se0, se1)
        ssem = (ss0, ss1, ss2)
        gops = [None, None, None]
        eops = [None, None]
        sops = [None, None, None]

        def issue_gather(c):
            q = c % 3
            gops[q] = pltpu.async_copy(
                table_hbm.at[idx_v.at[pl.ds(c * C, C)]], rows[q], gsem[q])

        def issue_enc(c):
            p = c & 1
            eops[p] = pltpu.async_copy(
                enc_hbm.at[pl.ds(col + c * C, C)], encs[p], esem[p])

        issue_gather(0)
        issue_enc(0)
        issue_gather(1)
        issue_enc(1)
        for c in range(n_chunks):
            q = c % 3
            p = c & 1
            gops[q].wait()
            eops[p].wait()

            def row_body(r, _, q=q, p=p):
                for j in range(D_MODEL // LANES):
                    o = j * LANES
                    rows[q][r, pl.ds(o, LANES)] = (
                        rows[q][r, pl.ds(o, LANES)]
                        + encs[p][r, pl.ds(o, LANES)])
                return 0

            lax.fori_loop(0, C, row_body, 0)
            sops[q] = pltpu.async_copy(
                rows[q], out_hbm.at[b, pl.ds(col + c * C, C)], ssem[q])
            if c + 2 < n_chunks:
                if c >= 1:
                    sops[(c - 1) % 3].wait()
                issue_gather(c + 2)
                issue_enc(c + 2)
        sops[(n_chunks - 3) % 3].wait()
        sops[(n_chunks - 2) % 3].wait()
        sops[(n_chunks - 1) % 3].wait()

    return emb_kernel


def kernel(x, table):
    batch, seq = x.shape
    enc = jnp.asarray(_pos_enc(seq))
    return _make_kernel(batch, seq)(x.astype(jnp.int32), table, enc)


# speedup vs baseline: 1.7169x; 1.0093x over previous
import functools

import jax
import jax.numpy as jnp
import numpy as np
from jax import lax
from jax.experimental import pallas as pl
from jax.experimental.pallas import tpu as pltpu
from jax.experimental.pallas import tpu_sc as plsc

D_MODEL = 768
LANES = 16


@functools.lru_cache(maxsize=None)
def _pos_enc(seq_len):
    pos = np.arange(seq_len, dtype=np.float32)[:, None]
    _2i = np.arange(0, D_MODEL, 2, dtype=np.float32)
    enc = np.zeros((seq_len, D_MODEL), dtype=np.float32)
    enc[:, 0::2] = np.sin(pos / np.float32(10000.0) ** (_2i / D_MODEL))
    enc[:, 1::2] = np.cos(pos / np.float32(10000.0) ** (_2i / D_MODEL))
    return enc


@functools.lru_cache(maxsize=None)
def _make_kernel(batch, seq):
    B = batch * seq
    info = plsc.get_sparse_core_info()
    NC, NS = info.num_cores, info.num_subcores
    NW = NC * NS
    b_per_w = B // NW
    C = 32
    n_chunks = b_per_w // C

    mesh = plsc.VectorSubcoreMesh(core_axis_name="c", subcore_axis_name="s")

    @functools.partial(
        pl.kernel,
        mesh=mesh,
        out_type=jax.ShapeDtypeStruct((batch, seq, D_MODEL), jnp.float32),
        scratch_types=[
            pltpu.VMEM((b_per_w,), jnp.int32),
            pltpu.VMEM((C, D_MODEL), jnp.float32),
            pltpu.VMEM((C, D_MODEL), jnp.float32),
            pltpu.VMEM((C, D_MODEL), jnp.float32),
            pltpu.VMEM((C, D_MODEL), jnp.float32),
            pltpu.SemaphoreType.DMA,
            pltpu.SemaphoreType.DMA,
            pltpu.SemaphoreType.DMA,
            pltpu.SemaphoreType.DMA,
            pltpu.SemaphoreType.DMA,
            pltpu.SemaphoreType.DMA,
        ],
    )
    def emb_kernel(idx_hbm, table_hbm, enc_hbm, out_hbm,
                   idx_v, rows0, rows1, enc0, enc1,
                   sg0, sg1, se0, se1, ss0, ss1):
        wid = lax.axis_index("s") * NC + lax.axis_index("c")
        base = wid * b_per_w
        b = base // seq
        col = base % seq
        pltpu.sync_copy(idx_hbm.at[b, pl.ds(col, b_per_w)], idx_v)

        rows = (rows0, rows1)
        encs = (enc0, enc1)
        gsem = (sg0, sg1)
        esem = (se0, se1)
        ssem = (ss0, ss1)
        gops = [None, None]
        eops = [None, None]
        sops = [None, None]

        def issue(c):
            p = c & 1
            gops[p] = pltpu.async_copy(
                table_hbm.at[idx_v.at[pl.ds(c * C, C)]], rows[p], gsem[p])
            eops[p] = pltpu.async_copy(
                enc_hbm.at[pl.ds(col + c * C, C)], encs[p], esem[p])

        issue(0)
        issue(1)
        for c in range(n_chunks):
            p = c & 1
            gops[p].wait()
            eops[p].wait()

            def row_body(r, _, p=p):
                for j in range(D_MODEL // LANES):
                    o = j * LANES
                    rows[p][r, pl.ds(o, LANES)] = (
                        rows[p][r, pl.ds(o, LANES)]
                        + encs[p][r, pl.ds(o, LANES)])
                return 0

            lax.fori_loop(0, C, row_body, 0)
            sops[p] = pltpu.async_copy(
                rows[p], out_hbm.at[b, pl.ds(col + c * C, C)], ssem[p])
            if c + 2 < n_chunks:
                sops[p].wait()
                issue(c + 2)
        sops[(n_chunks - 2) & 1].wait()
        sops[(n_chunks - 1) & 1].wait()

    return emb_kernel


def kernel(x, table):
    batch, seq = x.shape
    enc = jnp.asarray(_pos_enc(seq))
    return _make_kernel(batch, seq)(x.astype(jnp.int32), table, enc)
